# baseline (speedup 1.0000x reference)
import functools

import jax
import jax.numpy as jnp
from jax import lax
from jax.experimental import pallas as pl
from jax.experimental.pallas import tpu as pltpu
from jax.experimental.pallas import tpu_sc as plsc

VOCAB = 53
EMBED_DIM = 32
BATCH, SEQ = 16384, 200
TOTAL = BATCH * SEQ
NUM_WORKERS = 32
ROWS_PER_W = BATCH // NUM_WORKERS
ROWS_PER_CHUNK = 8
CHUNK = ROWS_PER_CHUNK * SEQ
NUM_CHUNKS = ROWS_PER_W // ROWS_PER_CHUNK
TAB = VOCAB * EMBED_DIM
WIN_OFFS = tuple(range(0, SEQ - 16, 16)) + (SEQ - 16,)

_mesh = plsc.VectorSubcoreMesh(core_axis_name="c", subcore_axis_name="s")


@functools.partial(
    pl.kernel,
    mesh=_mesh,
    out_type=jax.ShapeDtypeStruct((TOTAL * EMBED_DIM,), jnp.float32),
    scratch_types=[
        pltpu.VMEM((TAB,), jnp.float32),
        pltpu.VMEM((4, ROWS_PER_CHUNK, SEQ), jnp.int32),
        pltpu.VMEM((2, CHUNK * EMBED_DIM), jnp.float32),
        pltpu.SemaphoreType.DMA,
        pltpu.SemaphoreType.DMA,
        pltpu.SemaphoreType.DMA,
        pltpu.SemaphoreType.DMA,
        pltpu.SemaphoreType.DMA,
        pltpu.SemaphoreType.DMA,
    ],
    compiler_params=pltpu.CompilerParams(
        needs_layout_passes=False, disable_bounds_checks=True
    ),
)
def _embed_sc(
    idx_hbm, table_hbm, out_hbm, tab_v, idx_v, rows_v,
    sin0, sin1, sin2, sin3, sout0, sout1,
):
    wid = lax.axis_index("s") * 2 + lax.axis_index("c")
    pltpu.sync_copy(table_hbm, tab_v)
    lane = lax.iota(jnp.int32, 16)
    sins = (sin0, sin1, sin2, sin3)
    souts = (sout0, sout1)
    row0 = wid * ROWS_PER_W

    def in_slice(g):
        return idx_hbm.at[
            pl.ds(row0 + g * ROWS_PER_CHUNK, ROWS_PER_CHUNK), :
        ]

    def out_slice(g):
        return out_hbm.at[
            pl.ds((row0 + g * ROWS_PER_CHUNK) * SEQ * EMBED_DIM,
                  CHUNK * EMBED_DIM)
        ]

    for b in range(3):
        pltpu.async_copy(in_slice(b), idx_v.at[b], sins[b])

    def compute(ib, rb):
        def row_body(r, c):
            out_row = r * (SEQ * EMBED_DIM)
            for o in WIN_OFFS:
                iv = idx_v[ib, r, pl.ds(o, 16)]
                src = iv * EMBED_DIM
                ob = out_row + o * EMBED_DIM
                for k in range(16):
                    sel = jnp.full((16,), k, jnp.int32)
                    bc = jnp.take_along_axis(src, sel, axis=0)
                    a0 = bc + lane
                    v0 = plsc.load_gather(tab_v, [a0])
                    v1 = plsc.load_gather(tab_v, [a0 + 16])
                    rows_v[rb, pl.ds(ob + k * EMBED_DIM, 16)] = v0
                    rows_v[rb, pl.ds(ob + k * EMBED_DIM + 16, 16)] = v1
            return c

        lax.fori_loop(0, ROWS_PER_CHUNK, row_body, 0)

    def super_body(j, carry):
        for b in range(4):
            g = 4 * j + b
            rb = b % 2
            pltpu.make_async_copy(in_slice(0), idx_v.at[b], sins[b]).wait()

            @pl.when(g + 3 < NUM_CHUNKS)
            def _():
                pltpu.async_copy(
                    in_slice(g + 3), idx_v.at[(b + 3) % 4], sins[(b + 3) % 4]
                )

            @pl.when(g >= 2)
            def _():
                pltpu.make_async_copy(
                    rows_v.at[rb], out_slice(0), souts[rb]
                ).wait()

            compute(b, rb)
            pltpu.async_copy(rows_v.at[rb], out_slice(g), souts[rb])
        return carry

    lax.fori_loop(0, NUM_CHUNKS // 4, super_body, 0)
    pltpu.make_async_copy(rows_v.at[0], out_slice(0), souts[0]).wait()
    pltpu.make_async_copy(rows_v.at[1], out_slice(0), souts[1]).wait()


def kernel(card_indices, embedding_table):
    tab_flat = embedding_table.reshape(TAB)
    out = _embed_sc(card_indices.astype(jnp.int32), tab_flat)
    return out.reshape(BATCH, SEQ, EMBED_DIM)


# speedup vs baseline: 2.4695x; 2.4695x over previous
import functools

import jax
import jax.numpy as jnp
from jax import lax
from jax.experimental import pallas as pl
from jax.experimental.pallas import tpu as pltpu
from jax.experimental.pallas import tpu_sc as plsc

VOCAB = 53
EMBED_DIM = 32
BATCH, SEQ = 16384, 200
TOTAL = BATCH * SEQ
ROW_W = SEQ * EMBED_DIM
NUM_WORKERS = 32
ROWS_PER_W = BATCH // NUM_WORKERS
ROWS_PER_CHUNK = 8
CHUNK = ROWS_PER_CHUNK * SEQ
NUM_CHUNKS = ROWS_PER_W // ROWS_PER_CHUNK
SUPER = 4 * CHUNK
NUM_SUPERS = NUM_CHUNKS // 4
TAB = VOCAB * EMBED_DIM
WIN_OFFS = tuple(range(0, SEQ - 16, 16)) + (SEQ - 16,)

_mesh = plsc.VectorSubcoreMesh(core_axis_name="c", subcore_axis_name="s")


@functools.partial(
    pl.kernel,
    mesh=_mesh,
    out_type=jax.ShapeDtypeStruct((BATCH, ROW_W), jnp.float32),
    scratch_types=[
        pltpu.VMEM((TAB,), jnp.float32),
        pltpu.VMEM((2, SUPER), jnp.int32),
        pltpu.VMEM((2, ROWS_PER_CHUNK, ROW_W), jnp.float32),
        pltpu.SemaphoreType.DMA,
        pltpu.SemaphoreType.DMA,
        pltpu.SemaphoreType.DMA,
        pltpu.SemaphoreType.DMA,
    ],
    compiler_params=pltpu.CompilerParams(
        needs_layout_passes=False, disable_bounds_checks=True
    ),
)
def _embed_sc(
    idx_hbm, table_hbm, out_hbm, tab_v, idx_v, rows_v,
    sin0, sin1, sout0, sout1,
):
    wid = lax.axis_index("s") * 2 + lax.axis_index("c")
    pltpu.sync_copy(table_hbm, tab_v)
    lane = lax.iota(jnp.int32, 16)
    sins = (sin0, sin1)
    souts = (sout0, sout1)
    row0 = wid * ROWS_PER_W

    def in_slice(j):
        return idx_hbm.at[pl.ds((row0 + j * 4 * ROWS_PER_CHUNK) * SEQ, SUPER)]

    def out_slice(g):
        return out_hbm.at[
            pl.ds(row0 + g * ROWS_PER_CHUNK, ROWS_PER_CHUNK), :
        ]

    pltpu.async_copy(in_slice(0), idx_v.at[0], sins[0])

    def compute(jb, b, rb):
        def window(r, o, idx_row):
            iv = idx_v[jb, pl.ds(idx_row + o, 16)]
            src = iv * EMBED_DIM
            ob = o * EMBED_DIM
            for k in range(16):
                sel = jnp.full((16,), k, jnp.int32)
                bc = jnp.take_along_axis(src, sel, axis=0)
                a0 = bc + lane
                v0 = plsc.load_gather(tab_v, [a0])
                v1 = plsc.load_gather(tab_v, [a0 + 16])
                rows_v[rb, r, pl.ds(ob + k * EMBED_DIM, 16)] = v0
                rows_v[rb, r, pl.ds(ob + k * EMBED_DIM + 16, 16)] = v1

        def row_body(r, c):
            idx_row = b * CHUNK + r * SEQ

            def win_body(w, cc):
                window(r, w * 16, idx_row)
                return cc

            lax.fori_loop(0, SEQ // 16, win_body, 0)
            window(r, SEQ - 16, idx_row)
            return c

        lax.fori_loop(0, ROWS_PER_CHUNK, row_body, 0)

    def one_super(j, jb):
        pltpu.make_async_copy(in_slice(0), idx_v.at[jb], sins[jb]).wait()

        @pl.when(j + 1 < NUM_SUPERS)
        def _():
            pltpu.async_copy(in_slice(j + 1), idx_v.at[1 - jb], sins[1 - jb])

        for b in range(4):
            g = 4 * j + b
            rb = b % 2

            @pl.when(g >= 2)
            def _():
                pltpu.make_async_copy(
                    rows_v.at[rb], out_slice(0), souts[rb]
                ).wait()

            compute(jb, b, rb)
            pltpu.async_copy(rows_v.at[rb], out_slice(g), souts[rb])

    def super_pair(jj, carry):
        one_super(2 * jj, 0)
        one_super(2 * jj + 1, 1)
        return carry

    lax.fori_loop(0, NUM_SUPERS // 2, super_pair, 0)
    pltpu.make_async_copy(rows_v.at[0], out_slice(0), souts[0]).wait()
    pltpu.make_async_copy(rows_v.at[1], out_slice(0), souts[1]).wait()


def kernel(card_indices, embedding_table):
    idx_flat = card_indices.astype(jnp.int32).reshape(TOTAL)
    tab_flat = embedding_table.reshape(TAB)
    out = _embed_sc(idx_flat, tab_flat)
    return out.reshape(BATCH, SEQ, EMBED_DIM)


# speedup vs baseline: 2.5051x; 1.0144x over previous
import functools

import jax
import jax.numpy as jnp
from jax import lax
from jax.experimental import pallas as pl
from jax.experimental.pallas import tpu as pltpu
from jax.experimental.pallas import tpu_sc as plsc

VOCAB = 53
EMBED_DIM = 32
BATCH, SEQ = 16384, 200
TOTAL = BATCH * SEQ
ROW_W = SEQ * EMBED_DIM
NUM_WORKERS = 32
ROWS_PER_W = BATCH // NUM_WORKERS
ROWS_PER_CHUNK = 8
CHUNK = ROWS_PER_CHUNK * SEQ
NUM_CHUNKS = ROWS_PER_W // ROWS_PER_CHUNK
SUPER = 4 * CHUNK
NUM_SUPERS = NUM_CHUNKS // 4
TAB = VOCAB * EMBED_DIM
WIN_OFFS = tuple(range(0, SEQ - 16, 16)) + (SEQ - 16,)

_mesh = plsc.VectorSubcoreMesh(core_axis_name="c", subcore_axis_name="s")


@functools.partial(
    pl.kernel,
    mesh=_mesh,
    out_type=jax.ShapeDtypeStruct((BATCH, ROW_W), jnp.float32),
    scratch_types=[
        pltpu.VMEM((TAB,), jnp.float32),
        pltpu.VMEM((2, SUPER), jnp.int32),
        pltpu.VMEM((2, ROWS_PER_CHUNK, ROW_W), jnp.float32),
        pltpu.SemaphoreType.DMA,
        pltpu.SemaphoreType.DMA,
        pltpu.SemaphoreType.DMA,
        pltpu.SemaphoreType.DMA,
    ],
    compiler_params=pltpu.CompilerParams(
        needs_layout_passes=False, disable_bounds_checks=True
    ),
)
def _embed_sc(
    idx_hbm, table_hbm, out_hbm, tab_v, idx_v, rows_v,
    sin0, sin1, sout0, sout1,
):
    wid = lax.axis_index("s") * 2 + lax.axis_index("c")
    pltpu.sync_copy(table_hbm, tab_v)
    lane = lax.iota(jnp.int32, 16)
    sins = (sin0, sin1)
    souts = (sout0, sout1)
    row0 = wid * ROWS_PER_W

    def in_slice(j):
        return idx_hbm.at[pl.ds((row0 + j * 4 * ROWS_PER_CHUNK) * SEQ, SUPER)]

    def out_slice(g):
        return out_hbm.at[
            pl.ds(row0 + g * ROWS_PER_CHUNK, ROWS_PER_CHUNK), :
        ]

    pltpu.async_copy(in_slice(0), idx_v.at[0], sins[0])

    def compute(jb, b, rb):
        def gathered(iv, k):
            src = iv * EMBED_DIM
            sel = jnp.full((16,), k, jnp.int32)
            bc = jnp.take_along_axis(src, sel, axis=0)
            a0 = bc + lane
            return (
                plsc.load_gather(tab_v, [a0]),
                plsc.load_gather(tab_v, [a0 + 16]),
            )

        def pair_body(p, c):
            pbase = b * CHUNK + p * (2 * SEQ)
            r0 = 2 * p
            r1 = r0 + 1

            def win_lo(w, cc):
                iv = idx_v[jb, pl.ds(pbase + w * 16, 16)]
                for k in range(16):
                    v0, v1 = gathered(iv, k)
                    ob = w * 512 + k * EMBED_DIM
                    rows_v[rb, r0, pl.ds(ob, 16)] = v0
                    rows_v[rb, r0, pl.ds(ob + 16, 16)] = v1
                return cc

            lax.fori_loop(0, 12, win_lo, 0)

            iv_mid = idx_v[jb, pl.ds(pbase + 192, 16)]
            for k in range(16):
                v0, v1 = gathered(iv_mid, k)
                if k < 8:
                    ob = 6144 + k * EMBED_DIM
                    rows_v[rb, r0, pl.ds(ob, 16)] = v0
                    rows_v[rb, r0, pl.ds(ob + 16, 16)] = v1
                else:
                    ob = (k - 8) * EMBED_DIM
                    rows_v[rb, r1, pl.ds(ob, 16)] = v0
                    rows_v[rb, r1, pl.ds(ob + 16, 16)] = v1

            def win_hi(w, cc):
                iv = idx_v[jb, pl.ds(pbase + 208 + w * 16, 16)]
                for k in range(16):
                    v0, v1 = gathered(iv, k)
                    ob = 256 + w * 512 + k * EMBED_DIM
                    rows_v[rb, r1, pl.ds(ob, 16)] = v0
                    rows_v[rb, r1, pl.ds(ob + 16, 16)] = v1
                return cc

            lax.fori_loop(0, 12, win_hi, 0)
            return c

        lax.fori_loop(0, ROWS_PER_CHUNK // 2, pair_body, 0)

    def one_super(j, jb):
        pltpu.make_async_copy(in_slice(0), idx_v.at[jb], sins[jb]).wait()

        @pl.when(j + 1 < NUM_SUPERS)
        def _():
            pltpu.async_copy(in_slice(j + 1), idx_v.at[1 - jb], sins[1 - jb])

        for b in range(4):
            g = 4 * j + b
            rb = b % 2

            @pl.when(g >= 2)
            def _():
                pltpu.make_async_copy(
                    rows_v.at[rb], out_slice(0), souts[rb]
                ).wait()

            compute(jb, b, rb)
            pltpu.async_copy(rows_v.at[rb], out_slice(g), souts[rb])

    def super_pair(jj, carry):
        one_super(2 * jj, 0)
        one_super(2 * jj + 1, 1)
        return carry

    lax.fori_loop(0, NUM_SUPERS // 2, super_pair, 0)
    for rb in range(2):
        pltpu.make_async_copy(rows_v.at[rb], out_slice(0), souts[rb]).wait()


def kernel(card_indices, embedding_table):
    idx_flat = card_indices.astype(jnp.int32).reshape(TOTAL)
    tab_flat = embedding_table.reshape(TAB)
    out = _embed_sc(idx_flat, tab_flat)
    return out.reshape(BATCH, SEQ, EMBED_DIM)


# speedup vs baseline: 2.5447x; 1.0158x over previous
import functools

import jax
import jax.numpy as jnp
from jax import lax
from jax.experimental import pallas as pl
from jax.experimental.pallas import tpu as pltpu
from jax.experimental.pallas import tpu_sc as plsc

VOCAB = 53
EMBED_DIM = 32
BATCH, SEQ = 16384, 200
TOTAL = BATCH * SEQ
ROW_W = SEQ * EMBED_DIM
NUM_WORKERS = 32
ROWS_PER_W = BATCH // NUM_WORKERS
ROWS_PER_CHUNK = 8
CHUNK = ROWS_PER_CHUNK * SEQ
NUM_CHUNKS = ROWS_PER_W // ROWS_PER_CHUNK
SUPER = 4 * CHUNK
NUM_SUPERS = NUM_CHUNKS // 4
TAB = VOCAB * EMBED_DIM
WIN_OFFS = tuple(range(0, SEQ - 16, 16)) + (SEQ - 16,)

_mesh = plsc.VectorSubcoreMesh(core_axis_name="c", subcore_axis_name="s")


@functools.partial(
    pl.kernel,
    mesh=_mesh,
    out_type=jax.ShapeDtypeStruct((BATCH, ROW_W), jnp.float32),
    scratch_types=[
        pltpu.VMEM((TAB,), jnp.float32),
        pltpu.VMEM((2, 4 * ROWS_PER_CHUNK, SEQ), jnp.int32),
        pltpu.VMEM((2, ROWS_PER_CHUNK, ROW_W), jnp.float32),
        pltpu.SemaphoreType.DMA,
        pltpu.SemaphoreType.DMA,
        pltpu.SemaphoreType.DMA,
        pltpu.SemaphoreType.DMA,
    ],
    compiler_params=pltpu.CompilerParams(
        needs_layout_passes=False, disable_bounds_checks=True
    ),
)
def _embed_sc(
    idx_hbm, table_hbm, out_hbm, tab_v, idx_v, rows_v,
    sin0, sin1, sout0, sout1,
):
    wid = lax.axis_index("s") * 2 + lax.axis_index("c")
    pltpu.sync_copy(table_hbm, tab_v)
    lane = lax.iota(jnp.int32, 16)
    sins = (sin0, sin1)
    souts = (sout0, sout1)
    row0 = wid * ROWS_PER_W

    def in_slice(j):
        return idx_hbm.at[
            pl.ds(row0 + j * 4 * ROWS_PER_CHUNK, 4 * ROWS_PER_CHUNK), :
        ]

    def out_slice(g):
        return out_hbm.at[
            pl.ds(row0 + g * ROWS_PER_CHUNK, ROWS_PER_CHUNK), :
        ]

    pltpu.async_copy(in_slice(0), idx_v.at[0], sins[0])

    def compute(jb, b, rb):
        def window(rrow, r, o):
            iv = idx_v[jb, rrow, pl.ds(o, 16)]
            src = iv * EMBED_DIM
            for k in range(16):
                sel = jnp.full((16,), k, jnp.int32)
                bc = jnp.take_along_axis(src, sel, axis=0)
                a0 = bc + lane
                v0 = plsc.load_gather(tab_v, [a0])
                v1 = plsc.load_gather(tab_v, [a0 + 16])
                ob = o * EMBED_DIM + k * EMBED_DIM
                rows_v[rb, r, pl.ds(ob, 16)] = v0
                rows_v[rb, r, pl.ds(ob + 16, 16)] = v1

        def row_body(r, c):
            rrow = b * ROWS_PER_CHUNK + r

            def win_body(w, cc):
                window(rrow, r, w * 16)
                return cc

            lax.fori_loop(0, SEQ // 16, win_body, 0)
            window(rrow, r, SEQ - 16)
            return c

        lax.fori_loop(0, ROWS_PER_CHUNK, row_body, 0)

    def one_super(j, jb):
        pltpu.make_async_copy(in_slice(0), idx_v.at[jb], sins[jb]).wait()

        @pl.when(j + 1 < NUM_SUPERS)
        def _():
            pltpu.async_copy(in_slice(j + 1), idx_v.at[1 - jb], sins[1 - jb])

        for b in range(4):
            g = 4 * j + b
            rb = b % 2

            @pl.when(g >= 2)
            def _():
                pltpu.make_async_copy(
                    rows_v.at[rb], out_slice(0), souts[rb]
                ).wait()

            compute(jb, b, rb)
            pltpu.async_copy(rows_v.at[rb], out_slice(g), souts[rb])

    def super_pair(jj, carry):
        one_super(2 * jj, 0)
        one_super(2 * jj + 1, 1)
        return carry

    lax.fori_loop(0, NUM_SUPERS // 2, super_pair, 0)
    for rb in range(2):
        pltpu.make_async_copy(rows_v.at[rb], out_slice(0), souts[rb]).wait()


def kernel(card_indices, embedding_table):
    tab_flat = embedding_table.reshape(TAB)
    out = _embed_sc(card_indices.astype(jnp.int32), tab_flat)
    return out.reshape(BATCH, SEQ, EMBED_DIM)
